# baseline (device time: 21341 ns/iter reference)
import jax
import jax.numpy as jnp
from jax import lax
from jax.experimental import pallas as pl
from jax.experimental.pallas import tpu as pltpu

P = 16


def kernel(x):
    m, n = x.shape
    c = m // P

    def body(x_ref, out_ref, xb_ref, red_ref, rs_buf,
             rs_send, rs_recv, ag_send, ag_recv):
        my = lax.axis_index("i")

        barrier = pltpu.get_barrier_semaphore()
        for k in range(1, P):
            pl.semaphore_signal(
                barrier, inc=1,
                device_id=((my + k) % P,),
                device_id_type=pl.DeviceIdType.MESH,
            )
        pl.semaphore_wait(barrier, P - 1)

        xb_ref[...] = x_ref[...].astype(jnp.bfloat16)

        sends1 = []
        for k in range(1, P):
            dst = (my + k) % P
            rdma = pltpu.make_async_remote_copy(
                src_ref=xb_ref.at[pl.ds(dst * c, c), :],
                dst_ref=rs_buf.at[pl.ds(my * c, c), :],
                send_sem=rs_send.at[dst],
                recv_sem=rs_recv.at[my],
                device_id=(dst,),
                device_id_type=pl.DeviceIdType.MESH,
            )
            rdma.start()
            sends1.append(rdma)

        acc = x_ref[pl.ds(my * c, c), :]
        for k in range(1, P):
            src = (my - k) % P
            recv = pltpu.make_async_remote_copy(
                src_ref=xb_ref.at[pl.ds(0, c), :],
                dst_ref=rs_buf.at[pl.ds(src * c, c), :],
                send_sem=rs_send.at[src],
                recv_sem=rs_recv.at[src],
                device_id=(src,),
                device_id_type=pl.DeviceIdType.MESH,
            )
            recv.wait_recv()
            acc = acc + rs_buf[pl.ds(src * c, c), :].astype(jnp.float32)

        red_ref[...] = acc.astype(jnp.bfloat16)
        out_ref[pl.ds(my * c, c), :] = red_ref[...]

        sends2 = []
        for k in range(1, P):
            dst = (my + k) % P
            rdma = pltpu.make_async_remote_copy(
                src_ref=red_ref,
                dst_ref=out_ref.at[pl.ds(my * c, c), :],
                send_sem=ag_send.at[dst],
                recv_sem=ag_recv.at[my],
                device_id=(dst,),
                device_id_type=pl.DeviceIdType.MESH,
            )
            rdma.start()
            sends2.append(rdma)

        for k in range(1, P):
            src = (my - k) % P
            recv = pltpu.make_async_remote_copy(
                src_ref=red_ref,
                dst_ref=out_ref.at[pl.ds(src * c, c), :],
                send_sem=ag_send.at[src],
                recv_sem=ag_recv.at[src],
                device_id=(src,),
                device_id_type=pl.DeviceIdType.MESH,
            )
            recv.wait_recv()

        for r in sends1:
            r.wait_send()
        for r in sends2:
            r.wait_send()

    return pl.pallas_call(
        body,
        out_shape=jax.ShapeDtypeStruct((m, n), jnp.bfloat16),
        in_specs=[pl.BlockSpec(memory_space=pltpu.VMEM)],
        out_specs=pl.BlockSpec(memory_space=pltpu.VMEM),
        scratch_shapes=[
            pltpu.VMEM((m, n), jnp.bfloat16),
            pltpu.VMEM((c, n), jnp.bfloat16),
            pltpu.VMEM((m, n), jnp.bfloat16),
            pltpu.SemaphoreType.DMA((P,)),
            pltpu.SemaphoreType.DMA((P,)),
            pltpu.SemaphoreType.DMA((P,)),
            pltpu.SemaphoreType.DMA((P,)),
        ],
        compiler_params=pltpu.CompilerParams(collective_id=0),
    )(x)
